# baseline (speedup 1.0000x reference)
import jax
import jax.numpy as jnp
from jax.experimental import pallas as pl
from jax.experimental.pallas import tpu as pltpu
from jax.experimental.pallas import tpu_sc as plsc

_BATCH = 16384
_IN = 128
_OUT = 64
_GATHER_WINDOW = 128


def _sc_gather(emb_wide, idx2d):
    mesh = plsc.VectorSubcoreMesh(core_axis_name="core", subcore_axis_name="subcore")

    @pl.kernel(
        out_type=jax.ShapeDtypeStruct((_BATCH, 2 * _OUT), jnp.float32),
        mesh=mesh,
    )
    def gather_kernel(emb_hbm, i_hbm, o_hbm):
        def body(i_vmem, o_vmem):
            pltpu.sync_copy(emb_hbm.at[i_vmem.at[0]], o_vmem)

        pltpu.emit_pipeline(
            body,
            grid=(_BATCH // _GATHER_WINDOW,),
            in_specs=[
                pl.BlockSpec((1, _GATHER_WINDOW), index_map=lambda i: (0, i))
            ],
            out_specs=[
                pl.BlockSpec((_GATHER_WINDOW, 2 * _OUT), index_map=lambda i: (i, 0))
            ],
            core_axis_name=("core", "subcore"),
            dimension_semantics=(pltpu.PARALLEL,),
        )(i_hbm, o_hbm)

    return gather_kernel(emb_wide, idx2d)


def _tc_fused(x, W_fc, g, parity):
    blk = 2048

    def body(x_ref, wfc_ref, g_ref, p_ref, o_ref):
        h = jax.lax.dot_general(
            x_ref[...],
            wfc_ref[...],
            (((1,), (1,)), ((), ())),
            preferred_element_type=jnp.float32,
        )
        g = g_ref[...]
        w_i = jnp.where(p_ref[...] == 0, g[:, :_OUT], g[:, _OUT:])
        o_ref[...] = h * w_i

    return pl.pallas_call(
        body,
        grid=(_BATCH // blk,),
        in_specs=[
            pl.BlockSpec((blk, _IN), lambda i: (i, 0)),
            pl.BlockSpec((_OUT, _IN), lambda i: (0, 0)),
            pl.BlockSpec((blk, 2 * _OUT), lambda i: (i, 0)),
            pl.BlockSpec((blk, 1), lambda i: (i, 0)),
        ],
        out_specs=pl.BlockSpec((blk, _OUT), lambda i: (i, 0)),
        out_shape=jax.ShapeDtypeStruct((_BATCH, _OUT), jnp.float32),
    )(x, W_fc, g, parity)


def kernel(x, id, W_fc, emb):
    idx = id.astype(jnp.int32)
    emb_wide = emb.reshape(emb.shape[0] // 2, 2 * _OUT)
    gather_idx = (idx // 2).reshape(1, _BATCH)
    parity = (idx & 1).reshape(_BATCH, 1)
    g = _sc_gather(emb_wide, gather_idx)
    return _tc_fused(x, W_fc, g, parity)


# speedup vs baseline: 1.0021x; 1.0021x over previous
import jax
import jax.numpy as jnp
from jax import lax
from jax.experimental import pallas as pl
from jax.experimental.pallas import tpu as pltpu
from jax.experimental.pallas import tpu_sc as plsc

_BATCH = 16384
_IN = 128
_OUT = 64
_NC = 2
_NS = 16
_NW = _NC * _NS
_BPW = _BATCH // _NW


def _sc_gather(emb, idx):
    mesh = plsc.VectorSubcoreMesh(core_axis_name="c", subcore_axis_name="s")

    @pl.kernel(
        out_type=jax.ShapeDtypeStruct((_BATCH, _OUT), jnp.float32),
        mesh=mesh,
        scratch_types=[
            pltpu.VMEM((_BPW,), jnp.int32),
            pltpu.VMEM((_BPW, _OUT), jnp.float32),
            pltpu.SemaphoreType.DMA,
        ],
        compiler_params=pltpu.CompilerParams(use_tc_tiling_on_sc=False),
    )
    def gather_kernel(emb_hbm, idx_hbm, out_hbm, idx_v, rows_v, sem):
        wid = lax.axis_index("s") * _NC + lax.axis_index("c")
        base = wid * _BPW
        pltpu.sync_copy(idx_hbm.at[pl.ds(base, _BPW)], idx_v)
        pltpu.async_copy(emb_hbm.at[idx_v], rows_v, sem).wait()
        pltpu.sync_copy(rows_v, out_hbm.at[pl.ds(base, _BPW)])

    return gather_kernel(emb, idx)


def _tc_fused(x, W_fc, w):
    blk = 2048

    def body(x_ref, wfc_ref, w_ref, o_ref):
        h = jax.lax.dot_general(
            x_ref[...],
            wfc_ref[...],
            (((1,), (1,)), ((), ())),
            preferred_element_type=jnp.float32,
        )
        o_ref[...] = h * w_ref[...]

    return pl.pallas_call(
        body,
        grid=(_BATCH // blk,),
        in_specs=[
            pl.BlockSpec((blk, _IN), lambda i: (i, 0)),
            pl.BlockSpec((_OUT, _IN), lambda i: (0, 0)),
            pl.BlockSpec((blk, _OUT), lambda i: (i, 0)),
        ],
        out_specs=pl.BlockSpec((blk, _OUT), lambda i: (i, 0)),
        out_shape=jax.ShapeDtypeStruct((_BATCH, _OUT), jnp.float32),
    )(x, W_fc, w)


def kernel(x, id, W_fc, emb):
    idx = id.astype(jnp.int32)
    w_i = _sc_gather(emb, idx)
    return _tc_fused(x, W_fc, w_i)
